# baseline (speedup 1.0000x reference)
import functools

import jax
import jax.numpy as jnp
from jax import lax
from jax.experimental import pallas as pl
from jax.experimental.pallas import tpu as pltpu
from jax.experimental.pallas import tpu_sc as plsc

H = 128
GROUPS = 16
PER_GROUP = 8
SLOTS = GROUPS * PER_GROUP
B = 4096
ROW = SLOTS * H
OUT = GROUPS * H

NC, NS, L = 2, 16, 16
NW = NC * NS
ROWS_PER_W = B // NW
CR = 8
NQ = 8
CC = ROW // NQ
SPQ = SLOTS // NQ
JPQ = SPQ // GROUPS
NGR = ROWS_PER_W // CR
NBUF = 4

_MESH = plsc.VectorSubcoreMesh(core_axis_name="c", subcore_axis_name="s")


def _fold_chunk(buf, acc):
    @pl.loop(0, GROUPS)
    def _group(i):
        for hp in range(H // L):
            off = i * H + hp * L
            a = acc[pl.ds(off, L)]
            for r in range(CR):
                a = jnp.maximum(a, buf.at[r][pl.ds(off, L)])
            acc[pl.ds(off, L)] = a


@functools.partial(
    pl.kernel,
    out_type=jax.ShapeDtypeStruct((NW, OUT), jnp.float32),
    mesh=_MESH,
    scratch_types=[
        pltpu.VMEM((NBUF, CR, CC), jnp.float32),
        pltpu.VMEM((OUT,), jnp.float32),
        [pltpu.SemaphoreType.DMA] * NBUF,
    ],
)
def _stage1(x_hbm, part_hbm, bufs, acc, sems):
    wid = lax.axis_index("s") * NC + lax.axis_index("c")
    row0 = wid * ROWS_PER_W

    neg = jnp.full((L,), -jnp.inf, jnp.float32)

    @pl.loop(0, OUT // L)
    def _init(p):
        acc[pl.ds(p * L, L)] = neg

    def _start(gr, e, b):
        pltpu.async_copy(
            x_hbm.at[pl.ds(row0 + gr * CR, CR), pl.ds(e * CC, CC)],
            bufs.at[b],
            sems[b],
        )

    def _wait(b):
        pltpu.make_async_copy(
            x_hbm.at[pl.ds(0, CR), pl.ds(0, CC)], bufs.at[b], sems[b]
        ).wait()

    for b in range(NBUF):
        _start(0, b, b)

    @pl.loop(0, NGR)
    def _main(gr):
        for e in range(NQ):
            b = e % NBUF
            if e + NBUF < NQ:
                _wait(b)
                _fold_chunk(bufs.at[b], acc)
                _start(gr, e + NBUF, b)
            else:

                @pl.when(gr + 1 < NGR)
                def _prefetch():
                    _wait(b)
                    _fold_chunk(bufs.at[b], acc)
                    _start(gr + 1, e + NBUF - NQ, b)

                @pl.when(gr + 1 >= NGR)
                def _last():
                    _wait(b)
                    _fold_chunk(bufs.at[b], acc)

    pltpu.sync_copy(acc, part_hbm.at[wid])


def _stage2_body(p_ref, o_ref):
    o_ref[...] = jnp.max(p_ref[...], axis=0, keepdims=True)


def kernel(x):
    parts = _stage1(x)
    return pl.pallas_call(
        _stage2_body,
        out_shape=jax.ShapeDtypeStruct((1, OUT), jnp.float32),
    )(parts)


# speedup vs baseline: 1.6888x; 1.6888x over previous
import functools

import jax
import jax.numpy as jnp
from jax import lax
from jax.experimental import pallas as pl
from jax.experimental.pallas import tpu as pltpu
from jax.experimental.pallas import tpu_sc as plsc

H = 128
GROUPS = 16
PER_GROUP = 8
SLOTS = GROUPS * PER_GROUP
B = 4096
ROW = SLOTS * H
OUT = GROUPS * H

BT = 2304
BSC = B - BT
BR = 128
NBLK = BT // BR

NC, NS, L = 2, 16, 16
NW = NC * NS
ROWS_PER_W = BSC // NW
CR = 8
NQ = 4
CC = ROW // NQ
NGR = ROWS_PER_W // CR

_MESH = plsc.VectorSubcoreMesh(core_axis_name="c", subcore_axis_name="s")


def _fold_chunk(buf, acc):
    @pl.loop(0, GROUPS)
    def _group(i):
        for hp in range(H // L):
            off = i * H + hp * L
            a = acc[pl.ds(off, L)]
            for r in range(CR):
                row = buf.at[r]
                for jj in range(CC // (GROUPS * H)):
                    a = jnp.maximum(a, row[pl.ds(jj * GROUPS * H + off, L)])
            acc[pl.ds(off, L)] = a


@functools.partial(
    pl.kernel,
    out_type=jax.ShapeDtypeStruct((NW, OUT), jnp.float32),
    mesh=_MESH,
    scratch_types=[
        pltpu.VMEM((CR, CC), jnp.float32),
        pltpu.VMEM((CR, CC), jnp.float32),
        pltpu.VMEM((OUT,), jnp.float32),
        pltpu.SemaphoreType.DMA,
        pltpu.SemaphoreType.DMA,
    ],
)
def _sc_stage(x_hbm, part_hbm, buf0, buf1, acc, sem0, sem1):
    wid = lax.axis_index("s") * NC + lax.axis_index("c")
    row0 = BT + wid * ROWS_PER_W
    bufs = (buf0, buf1)
    sems = (sem0, sem1)

    neg = jnp.full((L,), -jnp.inf, jnp.float32)

    @pl.loop(0, OUT // L)
    def _init(p):
        acc[pl.ds(p * L, L)] = neg

    def _start(gr, q, b):
        pltpu.async_copy(
            x_hbm.at[pl.ds(row0 + gr * CR, CR), pl.ds(q * CC, CC)],
            bufs[b],
            sems[b],
        )

    def _wait(b):
        pltpu.make_async_copy(
            x_hbm.at[pl.ds(0, CR), pl.ds(0, CC)], bufs[b], sems[b]
        ).wait()

    _start(0, 0, 0)

    @pl.loop(0, NGR)
    def _main(gr):
        for q in range(NQ):
            b = q % 2
            nb = (q + 1) % 2
            if q < NQ - 1:
                _start(gr, q + 1, nb)
            else:

                @pl.when(gr + 1 < NGR)
                def _prefetch():
                    _start(gr + 1, 0, nb)

            _wait(b)
            _fold_chunk(bufs[b], acc)

    pltpu.sync_copy(acc, part_hbm.at[wid])


def _tc_body(x_ref, o_ref):
    o_ref[...] = jnp.max(x_ref[...], axis=0, keepdims=True)[None]


def _combine_body(sc_ref, tc_ref, o_ref):
    t = jnp.max(tc_ref[...].reshape(NBLK, ROW), axis=0)
    t = jnp.max(t.reshape(PER_GROUP, GROUPS, H), axis=0)
    s = jnp.max(sc_ref[...], axis=0).reshape(GROUPS, H)
    o_ref[...] = jnp.maximum(t, s).reshape(1, OUT)


def kernel(x):
    sc_parts = _sc_stage(x)
    tc_parts = pl.pallas_call(
        _tc_body,
        grid=(NBLK,),
        in_specs=[pl.BlockSpec((BR, ROW), lambda i: (i, 0))],
        out_specs=pl.BlockSpec((1, 1, ROW), lambda i: (i, 0, 0)),
        out_shape=jax.ShapeDtypeStruct((NBLK, 1, ROW), jnp.float32),
    )(x)
    return pl.pallas_call(
        _combine_body,
        out_shape=jax.ShapeDtypeStruct((1, OUT), jnp.float32),
    )(sc_parts, tc_parts)


# speedup vs baseline: 2.1339x; 1.2635x over previous
import functools

import jax
import jax.numpy as jnp
from jax import lax
from jax.experimental import pallas as pl
from jax.experimental.pallas import tpu as pltpu
from jax.experimental.pallas import tpu_sc as plsc

H = 128
GROUPS = 16
PER_GROUP = 8
SLOTS = GROUPS * PER_GROUP
B = 4096
ROW = SLOTS * H
OUT = GROUPS * H

BT = 2304
BSC = B - BT
BR = 128
NBLK = BT // BR

NC, NS, L = 2, 16, 16
NW = NC * NS
ROWS_PER_W = BSC // NW
CR = 8
NQ = 4
CC = ROW // NQ
NGR = ROWS_PER_W // CR

_MESH = plsc.VectorSubcoreMesh(core_axis_name="c", subcore_axis_name="s")


def _fold_chunk(buf, acc):
    @pl.loop(0, GROUPS)
    def _group(i):
        for hp in range(H // L):
            off = i * H + hp * L
            a = acc[pl.ds(off, L)]
            for r in range(CR):
                row = buf.at[r]
                for jj in range(CC // (GROUPS * H)):
                    a = jnp.maximum(a, row[pl.ds(jj * GROUPS * H + off, L)])
            acc[pl.ds(off, L)] = a


@functools.partial(
    pl.kernel,
    out_type=jax.ShapeDtypeStruct((NW, OUT), jnp.float32),
    mesh=_MESH,
    scratch_types=[
        pltpu.VMEM((CR, CC), jnp.float32),
        pltpu.VMEM((CR, CC), jnp.float32),
        pltpu.VMEM((OUT,), jnp.float32),
        pltpu.SemaphoreType.DMA,
        pltpu.SemaphoreType.DMA,
    ],
)
def _sc_stage(x_hbm, part_hbm, buf0, buf1, acc, sem0, sem1):
    wid = lax.axis_index("s") * NC + lax.axis_index("c")
    row0 = BT + wid * ROWS_PER_W
    bufs = (buf0, buf1)
    sems = (sem0, sem1)

    neg = jnp.full((L,), -jnp.inf, jnp.float32)

    @pl.loop(0, OUT // L)
    def _init(p):
        acc[pl.ds(p * L, L)] = neg

    def _start(gr, q, b):
        pltpu.async_copy(
            x_hbm.at[pl.ds(row0 + gr * CR, CR), pl.ds(q * CC, CC)],
            bufs[b],
            sems[b],
        )

    def _wait(b):
        pltpu.make_async_copy(
            x_hbm.at[pl.ds(0, CR), pl.ds(0, CC)], bufs[b], sems[b]
        ).wait()

    _start(0, 0, 0)

    @pl.loop(0, NGR)
    def _main(gr):
        for q in range(NQ):
            b = q % 2
            nb = (q + 1) % 2
            if q < NQ - 1:
                _start(gr, q + 1, nb)
            else:

                @pl.when(gr + 1 < NGR)
                def _prefetch():
                    _start(gr + 1, 0, nb)

            _wait(b)
            _fold_chunk(bufs[b], acc)

    pltpu.sync_copy(acc, part_hbm.at[wid])


def _tc_body(x_ref, o_ref):
    o_ref[...] = jnp.max(x_ref[...], axis=0, keepdims=True)[None]


def _combine_body(sc_ref, tc_ref, o_ref):
    t = jnp.max(tc_ref[...].reshape(NBLK, ROW), axis=0)
    t = jnp.max(t.reshape(PER_GROUP, GROUPS, H), axis=0)
    s = jnp.max(sc_ref[...], axis=0).reshape(GROUPS, H)
    o_ref[...] = jnp.maximum(t, s).reshape(1, OUT)


def _combine_tc_only(tc_ref, o_ref):
    t = jnp.max(tc_ref[...].reshape(B // BR, ROW), axis=0)
    t = jnp.max(t.reshape(PER_GROUP, GROUPS, H), axis=0)
    o_ref[...] = t.reshape(1, OUT)


def kernel(x):
    tc_parts = pl.pallas_call(
        _tc_body,
        grid=(B // BR,),
        in_specs=[pl.BlockSpec((BR, ROW), lambda i: (i, 0))],
        out_specs=pl.BlockSpec((1, 1, ROW), lambda i: (i, 0, 0)),
        out_shape=jax.ShapeDtypeStruct((B // BR, 1, ROW), jnp.float32),
    )(x)
    return pl.pallas_call(
        _combine_tc_only,
        out_shape=jax.ShapeDtypeStruct((1, OUT), jnp.float32),
    )(tc_parts)
